# baseline (speedup 1.0000x reference)
import numpy as np
import jax
import jax.numpy as jnp
from jax import lax
from jax.experimental import pallas as pl
from jax.experimental.pallas import tpu as pltpu
from jax.experimental.pallas import tpu_sc as plsc

_B = 16384
_NC = 2
_NS = 16
_NW = _NC * _NS
_BPW = _B // _NW
_CHUNK = 128
_L = 16
_NG = _CHUNK // _L
_NJ3 = _BPW * 3 // _CHUNK
_NJ2 = _BPW * 2 // _CHUNK
_NJ1 = _BPW // _CHUNK

_CONSTS = np.array(
    [[(l + r) // 3 for l in range(_L)] for r in range(3)]
    + [[(l + r) % 3 for l in range(_L)] for r in range(3)]
    + [[l // 2 for l in range(_L)], [l % 2 for l in range(_L)]],
    dtype=np.int32)


def _gather_body(rot_hbm, tr_hbm, f_hbm, pp_hbm, idxf_hbm, cv_hbm,
                 r_out, t_out, ff_out, pps_out,
                 cv, idx_v, rep3, rep2, ridx3, ridx2, el3, el2,
                 r_v, t_v, f_v, pp_v, sem_rep, sem_dat):
    wid = lax.axis_index("s") * _NC + lax.axis_index("c")
    base = wid * _BPW

    pltpu.sync_copy(cv_hbm, cv)
    for j in range(_NJ1):
        pltpu.sync_copy(idxf_hbm.at[pl.ds(base + j * _CHUNK, _CHUNK)],
                        idx_v.at[j])

    dat = []
    for j in range(_NJ1):
        dat.append(pltpu.async_copy(
            f_hbm.at[idx_v.at[j]], f_v.at[pl.ds(j * _CHUNK, _CHUNK)], sem_dat))

    for G in range(_NJ3 * _NG):
        s = pl.ds((G % _NG) * _L, _L)
        rep3[G // _NG, s] = cv[G % 3] + (base + 5 * G + G // 3)
    for G in range(_NJ2 * _NG):
        s = pl.ds((G % _NG) * _L, _L)
        rep2[G // _NG, s] = cv[6] + (base + 8 * G)

    rep = []
    for j in range(_NJ3):
        rep.append(pltpu.async_copy(idxf_hbm.at[rep3.at[j]], ridx3.at[j], sem_rep))
    for j in range(_NJ2):
        rep.append(pltpu.async_copy(idxf_hbm.at[rep2.at[j]], ridx2.at[j], sem_rep))

    for j in range(_NJ3):
        rep[j].wait()
        for g in range(_NG):
            G = j * _NG + g
            s = pl.ds(g * _L, _L)
            el3[j, s] = ridx3[j, s] * 3 + cv[3 + G % 3]
        o = pl.ds(j * _CHUNK, _CHUNK)
        dat.append(pltpu.async_copy(rot_hbm.at[el3.at[j]], r_v.at[o], sem_dat))
        dat.append(pltpu.async_copy(tr_hbm.at[el3.at[j]], t_v.at[o], sem_dat))
    for j in range(_NJ2):
        rep[_NJ3 + j].wait()
        for g in range(_NG):
            s = pl.ds(g * _L, _L)
            el2[j, s] = ridx2[j, s] * 2 + cv[7]
        o = pl.ds(j * _CHUNK, _CHUNK)
        dat.append(pltpu.async_copy(pp_hbm.at[el2.at[j]], pp_v.at[o], sem_dat))
    for c in dat:
        c.wait()

    pltpu.sync_copy(r_v, r_out.at[pl.ds(base * 3, _BPW * 3)])
    pltpu.sync_copy(t_v, t_out.at[pl.ds(base * 3, _BPW * 3)])
    pltpu.sync_copy(f_v, ff_out.at[pl.ds(base, _BPW)])
    pltpu.sync_copy(pp_v, pps_out.at[pl.ds(base * 2, _BPW * 2)])


def kernel(rotvecs, translations, f, pp, camera_idxs):
    idxf = camera_idxs.astype(jnp.int32)
    mesh = plsc.VectorSubcoreMesh(core_axis_name="c", subcore_axis_name="s")
    run = pl.kernel(
        _gather_body,
        out_type=(
            jax.ShapeDtypeStruct((_B * 3,), jnp.float32),
            jax.ShapeDtypeStruct((_B * 3,), jnp.float32),
            jax.ShapeDtypeStruct((_B,), jnp.float32),
            jax.ShapeDtypeStruct((_B * 2,), jnp.float32),
        ),
        mesh=mesh,
        scratch_types=[
            pltpu.VMEM((8, _L), jnp.int32),
            pltpu.VMEM((_NJ1, _CHUNK), jnp.int32),
            pltpu.VMEM((_NJ3, _CHUNK), jnp.int32),
            pltpu.VMEM((_NJ2, _CHUNK), jnp.int32),
            pltpu.VMEM((_NJ3, _CHUNK), jnp.int32),
            pltpu.VMEM((_NJ2, _CHUNK), jnp.int32),
            pltpu.VMEM((_NJ3, _CHUNK), jnp.int32),
            pltpu.VMEM((_NJ2, _CHUNK), jnp.int32),
            pltpu.VMEM((_BPW * 3,), jnp.float32),
            pltpu.VMEM((_BPW * 3,), jnp.float32),
            pltpu.VMEM((_BPW,), jnp.float32),
            pltpu.VMEM((_BPW * 2,), jnp.float32),
            pltpu.SemaphoreType.DMA,
            pltpu.SemaphoreType.DMA,
        ],
    )
    r, t, ff, pps = run(rotvecs.reshape(-1), translations.reshape(-1), f,
                        pp.reshape(-1), idxf, jnp.asarray(_CONSTS))
    return (r.reshape(_B, 3), t.reshape(_B, 3), ff, pps.reshape(_B, 2))


# speedup vs baseline: 1.0027x; 1.0027x over previous
import numpy as np
import jax
import jax.numpy as jnp
from jax import lax
from jax.experimental import pallas as pl
from jax.experimental.pallas import tpu as pltpu
from jax.experimental.pallas import tpu_sc as plsc

_B = 16384
_NC = 2
_NS = 16
_NW = _NC * _NS
_BPW = _B // _NW
_CHUNK = 128
_L = 16
_NG = _CHUNK // _L
_NJ3 = _BPW * 3 // _CHUNK
_NJ2 = _BPW * 2 // _CHUNK
_NJ1 = _BPW // _CHUNK

_CONSTS = np.array(
    [[5 * r + (l + r) // 3 for l in range(_L)] for r in range(3)]
    + [[(l + r) % 3 for l in range(_L)] for r in range(3)]
    + [[l // 2 for l in range(_L)], [l % 2 for l in range(_L)]],
    dtype=np.int32)


def _gather_body(rot_hbm, tr_hbm, f_hbm, pp_hbm, idxf_hbm, cv_hbm,
                 r_out, t_out, ff_out, pps_out,
                 cv, idx_v, rep3, rep2, ridx3, ridx2, el3, el2,
                 r_v, t_v, f_v, pp_v, sem_rep, sem_dat):
    wid = lax.axis_index("s") * _NC + lax.axis_index("c")
    base = wid * _BPW

    pltpu.sync_copy(cv_hbm, cv)

    for j in range(_NJ1):
        pltpu.sync_copy(idxf_hbm.at[pl.ds(base + j * _CHUNK, _CHUNK)],
                        idx_v.at[j])
        pltpu.async_copy(
            f_hbm.at[idx_v.at[j]], f_v.at[pl.ds(j * _CHUNK, _CHUNK)], sem_dat)

    def rep3_body(J, carry):
        for bb in range(_NG):
            for r in range(3):
                c0, gm = divmod(3 * bb + r, _NG)
                rep3[3 * J + c0, pl.ds(gm * _L, _L)] = (
                    cv[r] + (base + 128 * J + 16 * bb))
        for c in range(3):
            pltpu.async_copy(idxf_hbm.at[rep3.at[3 * J + c]],
                             ridx3.at[pl.ds(384 * J + 128 * c, _CHUNK)],
                             sem_rep)
        return carry
    lax.fori_loop(0, _NJ3 // 3, rep3_body, 0)

    def rep2_body(j, carry):
        for g in range(_NG):
            rep2[j, pl.ds(g * _L, _L)] = cv[6] + (base + 64 * j + 8 * g)
        pltpu.async_copy(idxf_hbm.at[rep2.at[j]],
                         ridx2.at[pl.ds(128 * j, _CHUNK)], sem_rep)
        return carry
    lax.fori_loop(0, _NJ2, rep2_body, 0)

    pltpu.make_async_copy(idxf_hbm.at[pl.ds(0, _BPW * 3)], ridx3, sem_rep).wait()
    pltpu.make_async_copy(idxf_hbm.at[pl.ds(0, _BPW * 2)], ridx2, sem_rep).wait()

    def el3_body(J, carry):
        for c in range(3):
            row = 3 * J + c
            for g in range(_NG):
                m = (2 * c + g) % 3
                o = 384 * J + 128 * c + g * _L
                el3[row, pl.ds(g * _L, _L)] = (
                    ridx3[pl.ds(o, _L)] * 3 + cv[3 + m])
            o = pl.ds(384 * J + 128 * c, _CHUNK)
            pltpu.async_copy(rot_hbm.at[el3.at[row]], r_v.at[o], sem_dat)
            pltpu.async_copy(tr_hbm.at[el3.at[row]], t_v.at[o], sem_dat)
        return carry
    lax.fori_loop(0, _NJ3 // 3, el3_body, 0)

    def el2_body(j, carry):
        for g in range(_NG):
            el2[j, pl.ds(g * _L, _L)] = (
                ridx2[pl.ds(128 * j + g * _L, _L)] * 2 + cv[7])
        o = pl.ds(128 * j, _CHUNK)
        pltpu.async_copy(pp_hbm.at[el2.at[j]], pp_v.at[o], sem_dat)
        return carry
    lax.fori_loop(0, _NJ2, el2_body, 0)

    pltpu.make_async_copy(rot_hbm.at[pl.ds(0, _BPW * 3)], r_v, sem_dat).wait()
    pltpu.make_async_copy(tr_hbm.at[pl.ds(0, _BPW * 3)], t_v, sem_dat).wait()
    pltpu.make_async_copy(f_hbm.at[pl.ds(0, _BPW)], f_v, sem_dat).wait()
    pltpu.make_async_copy(pp_hbm.at[pl.ds(0, _BPW * 2)], pp_v, sem_dat).wait()

    pltpu.sync_copy(r_v, r_out.at[pl.ds(base * 3, _BPW * 3)])
    pltpu.sync_copy(t_v, t_out.at[pl.ds(base * 3, _BPW * 3)])
    pltpu.sync_copy(f_v, ff_out.at[pl.ds(base, _BPW)])
    pltpu.sync_copy(pp_v, pps_out.at[pl.ds(base * 2, _BPW * 2)])


def kernel(rotvecs, translations, f, pp, camera_idxs):
    idxf = camera_idxs.astype(jnp.int32)
    mesh = plsc.VectorSubcoreMesh(core_axis_name="c", subcore_axis_name="s")
    run = pl.kernel(
        _gather_body,
        out_type=(
            jax.ShapeDtypeStruct((_B * 3,), jnp.float32),
            jax.ShapeDtypeStruct((_B * 3,), jnp.float32),
            jax.ShapeDtypeStruct((_B,), jnp.float32),
            jax.ShapeDtypeStruct((_B * 2,), jnp.float32),
        ),
        mesh=mesh,
        scratch_types=[
            pltpu.VMEM((8, _L), jnp.int32),
            pltpu.VMEM((_NJ1, _CHUNK), jnp.int32),
            pltpu.VMEM((_NJ3, _CHUNK), jnp.int32),
            pltpu.VMEM((_NJ2, _CHUNK), jnp.int32),
            pltpu.VMEM((_BPW * 3,), jnp.int32),
            pltpu.VMEM((_BPW * 2,), jnp.int32),
            pltpu.VMEM((_NJ3, _CHUNK), jnp.int32),
            pltpu.VMEM((_NJ2, _CHUNK), jnp.int32),
            pltpu.VMEM((_BPW * 3,), jnp.float32),
            pltpu.VMEM((_BPW * 3,), jnp.float32),
            pltpu.VMEM((_BPW,), jnp.float32),
            pltpu.VMEM((_BPW * 2,), jnp.float32),
            pltpu.SemaphoreType.DMA,
            pltpu.SemaphoreType.DMA,
        ],
    )
    r, t, ff, pps = run(rotvecs.reshape(-1), translations.reshape(-1), f,
                        pp.reshape(-1), idxf, jnp.asarray(_CONSTS))
    return (r.reshape(_B, 3), t.reshape(_B, 3), ff, pps.reshape(_B, 2))


# speedup vs baseline: 7.0947x; 7.0755x over previous
import jax
import jax.numpy as jnp
from jax import lax
from jax.experimental import pallas as pl
from jax.experimental.pallas import tpu as pltpu
from jax.experimental.pallas import tpu_sc as plsc

_N = 100000
_B = 16384
_NC = 2
_NS = 16
_NW = _NC * _NS
_BPW = _B // _NW
_CHUNK = 128
_L = 16
_NG = _CHUNK // _L
_NJ = _BPW // _CHUNK


def _gather_body(rot_hbm, tr_hbm, f_hbm, pp_hbm, idx_hbm,
                 r_out, t_out, ff_out, pps_out,
                 idx_v, iy, iz, r_v, t_v, f_v, pp_v, sem):
    wid = lax.axis_index("s") * _NC + lax.axis_index("c")
    base = wid * _BPW

    for j in range(_NJ):
        pltpu.sync_copy(idx_hbm.at[pl.ds(base + j * _CHUNK, _CHUNK)],
                        idx_v.at[j])
    for j in range(_NJ):
        for g in range(_NG):
            s = pl.ds(g * _L, _L)
            v = idx_v[j, s]
            iy[j, s] = v + _N
            iz[j, s] = v + 2 * _N

    for j in range(_NJ):
        o = j * _CHUNK
        for ilist, cat, c in ((idx_v, r_v, 0), (iy, r_v, 1), (iz, r_v, 2)):
            pltpu.async_copy(rot_hbm.at[ilist.at[j]],
                             cat.at[pl.ds(c * _BPW + o, _CHUNK)], sem)
        for ilist, cat, c in ((idx_v, t_v, 0), (iy, t_v, 1), (iz, t_v, 2)):
            pltpu.async_copy(tr_hbm.at[ilist.at[j]],
                             cat.at[pl.ds(c * _BPW + o, _CHUNK)], sem)
        for ilist, c in ((idx_v, 0), (iy, 1)):
            pltpu.async_copy(pp_hbm.at[ilist.at[j]],
                             pp_v.at[pl.ds(c * _BPW + o, _CHUNK)], sem)
        pltpu.async_copy(f_hbm.at[idx_v.at[j]],
                         f_v.at[pl.ds(o, _CHUNK)], sem)

    pltpu.make_async_copy(rot_hbm.at[pl.ds(0, _BPW * 3)], r_v, sem).wait()
    pltpu.make_async_copy(tr_hbm.at[pl.ds(0, _BPW * 3)], t_v, sem).wait()
    pltpu.make_async_copy(f_hbm.at[pl.ds(0, _BPW)], f_v, sem).wait()
    pltpu.make_async_copy(pp_hbm.at[pl.ds(0, _BPW * 2)], pp_v, sem).wait()

    for c in range(3):
        pltpu.sync_copy(r_v.at[pl.ds(c * _BPW, _BPW)],
                        r_out.at[pl.ds(c * _B + base, _BPW)])
        pltpu.sync_copy(t_v.at[pl.ds(c * _BPW, _BPW)],
                        t_out.at[pl.ds(c * _B + base, _BPW)])
    for c in range(2):
        pltpu.sync_copy(pp_v.at[pl.ds(c * _BPW, _BPW)],
                        pps_out.at[pl.ds(c * _B + base, _BPW)])
    pltpu.sync_copy(f_v, ff_out.at[pl.ds(base, _BPW)])


def kernel(rotvecs, translations, f, pp, camera_idxs):
    idxf = camera_idxs.astype(jnp.int32)
    mesh = plsc.VectorSubcoreMesh(core_axis_name="c", subcore_axis_name="s")
    run = pl.kernel(
        _gather_body,
        out_type=(
            jax.ShapeDtypeStruct((_B * 3,), jnp.float32),
            jax.ShapeDtypeStruct((_B * 3,), jnp.float32),
            jax.ShapeDtypeStruct((_B,), jnp.float32),
            jax.ShapeDtypeStruct((_B * 2,), jnp.float32),
        ),
        mesh=mesh,
        scratch_types=[
            pltpu.VMEM((_NJ, _CHUNK), jnp.int32),
            pltpu.VMEM((_NJ, _CHUNK), jnp.int32),
            pltpu.VMEM((_NJ, _CHUNK), jnp.int32),
            pltpu.VMEM((_BPW * 3,), jnp.float32),
            pltpu.VMEM((_BPW * 3,), jnp.float32),
            pltpu.VMEM((_BPW,), jnp.float32),
            pltpu.VMEM((_BPW * 2,), jnp.float32),
            pltpu.SemaphoreType.DMA,
        ],
    )
    rT, tT, ff, pT = run(rotvecs.T.reshape(-1), translations.T.reshape(-1),
                         f, pp.T.reshape(-1), idxf)
    r = rT.reshape(3, _B).T
    t = tT.reshape(3, _B).T
    pps = pT.reshape(2, _B).T
    return (r, t, ff, pps)


# speedup vs baseline: 7.4429x; 1.0491x over previous
import jax
import jax.numpy as jnp
from jax import lax
from jax.experimental import pallas as pl
from jax.experimental.pallas import tpu as pltpu
from jax.experimental.pallas import tpu_sc as plsc

_N = 100000
_B = 16384
_NC = 2
_NS = 16
_NW = _NC * _NS
_BPW = _B // _NW
_CHUNK = 128
_L = 16
_NG = _CHUNK // _L
_NJ = _BPW // _CHUNK


def _gather_body(rot_hbm, tr_hbm, f_hbm, pp_hbm, idx_hbm,
                 r_out, t_out, ff_out, pps_out,
                 idx_v, iy, iz, r_v, t_v, f_v, pp_v, sem, sem_out):
    wid = lax.axis_index("s") * _NC + lax.axis_index("c")
    base = wid * _BPW

    stage = [pltpu.async_copy(idx_hbm.at[pl.ds(base + j * _CHUNK, _CHUNK)],
                              idx_v.at[j], sem)
             for j in range(_NJ)]
    for c in stage:
        c.wait()
    for j in range(_NJ):
        for g in range(_NG):
            s = pl.ds(g * _L, _L)
            v = idx_v[j, s]
            iy[j, s] = v + _N
            iz[j, s] = v + 2 * _N

    for j in range(_NJ):
        o = j * _CHUNK
        for ilist, cat, c in ((idx_v, r_v, 0), (iy, r_v, 1), (iz, r_v, 2)):
            pltpu.async_copy(rot_hbm.at[ilist.at[j]],
                             cat.at[pl.ds(c * _BPW + o, _CHUNK)], sem)
        for ilist, cat, c in ((idx_v, t_v, 0), (iy, t_v, 1), (iz, t_v, 2)):
            pltpu.async_copy(tr_hbm.at[ilist.at[j]],
                             cat.at[pl.ds(c * _BPW + o, _CHUNK)], sem)
        for ilist, c in ((idx_v, 0), (iy, 1)):
            pltpu.async_copy(pp_hbm.at[ilist.at[j]],
                             pp_v.at[pl.ds(c * _BPW + o, _CHUNK)], sem)
        pltpu.async_copy(f_hbm.at[idx_v.at[j]],
                         f_v.at[pl.ds(o, _CHUNK)], sem)

    pltpu.make_async_copy(rot_hbm.at[pl.ds(0, _BPW * 3)], r_v, sem).wait()
    pltpu.make_async_copy(tr_hbm.at[pl.ds(0, _BPW * 3)], t_v, sem).wait()
    pltpu.make_async_copy(f_hbm.at[pl.ds(0, _BPW)], f_v, sem).wait()
    pltpu.make_async_copy(pp_hbm.at[pl.ds(0, _BPW * 2)], pp_v, sem).wait()

    out = []
    for c in range(3):
        out.append(pltpu.async_copy(r_v.at[pl.ds(c * _BPW, _BPW)],
                                    r_out.at[pl.ds(c * _B + base, _BPW)],
                                    sem_out))
        out.append(pltpu.async_copy(t_v.at[pl.ds(c * _BPW, _BPW)],
                                    t_out.at[pl.ds(c * _B + base, _BPW)],
                                    sem_out))
    for c in range(2):
        out.append(pltpu.async_copy(pp_v.at[pl.ds(c * _BPW, _BPW)],
                                    pps_out.at[pl.ds(c * _B + base, _BPW)],
                                    sem_out))
    out.append(pltpu.async_copy(f_v, ff_out.at[pl.ds(base, _BPW)], sem_out))
    for c in out:
        c.wait()


def kernel(rotvecs, translations, f, pp, camera_idxs):
    idxf = camera_idxs.astype(jnp.int32)
    mesh = plsc.VectorSubcoreMesh(core_axis_name="c", subcore_axis_name="s")
    run = pl.kernel(
        _gather_body,
        out_type=(
            jax.ShapeDtypeStruct((_B * 3,), jnp.float32),
            jax.ShapeDtypeStruct((_B * 3,), jnp.float32),
            jax.ShapeDtypeStruct((_B,), jnp.float32),
            jax.ShapeDtypeStruct((_B * 2,), jnp.float32),
        ),
        mesh=mesh,
        scratch_types=[
            pltpu.VMEM((_NJ, _CHUNK), jnp.int32),
            pltpu.VMEM((_NJ, _CHUNK), jnp.int32),
            pltpu.VMEM((_NJ, _CHUNK), jnp.int32),
            pltpu.VMEM((_BPW * 3,), jnp.float32),
            pltpu.VMEM((_BPW * 3,), jnp.float32),
            pltpu.VMEM((_BPW,), jnp.float32),
            pltpu.VMEM((_BPW * 2,), jnp.float32),
            pltpu.SemaphoreType.DMA,
            pltpu.SemaphoreType.DMA,
        ],
    )
    rT, tT, ff, pT = run(rotvecs.T.reshape(-1), translations.T.reshape(-1),
                         f, pp.T.reshape(-1), idxf)
    r = rT.reshape(3, _B).T
    t = tT.reshape(3, _B).T
    pps = pT.reshape(2, _B).T
    return (r, t, ff, pps)
